# baseline (speedup 1.0000x reference)
import math

import jax
import jax.numpy as jnp
from jax import lax
from jax.experimental import pallas as pl
from jax.experimental.pallas import tpu as pltpu
from jax.experimental.pallas import tpu_sc as plsc

VOCAB = 1000000
D = 64
ROWS = 16384
COLS = 200

NC = 2
NS = 16
NW = NC * NS
ROWS_W = ROWS // NW
CR = 2
NCH = ROWS_W // CR
SCALE = math.sqrt(D)

_mesh = plsc.VectorSubcoreMesh(
    core_axis_name="c", subcore_axis_name="s", num_cores=NC, num_subcores=NS
)


def _body(table_hbm, x_hbm, out_hbm, idx0, idx1, rows0, rows1,
          gsem0, gsem1, ssem0, ssem1):
    wid = lax.axis_index("s") * NC + lax.axis_index("c")
    base = wid * ROWS_W
    idx = [idx0, idx1]
    rows = [rows0, rows1]
    gsem = [gsem0, gsem1]
    ssem = [ssem0, ssem1]

    def start_gather(ch, b):
        r0 = base + ch * CR
        pltpu.sync_copy(x_hbm.at[pl.ds(r0, CR)], idx[b])
        for j in range(CR):
            pltpu.async_copy(table_hbm.at[idx[b].at[j]], rows[b].at[j], gsem[b])

    def wait_gather(b):
        for j in range(CR):
            pltpu.make_async_copy(
                table_hbm.at[idx[b].at[j]], rows[b].at[j], gsem[b]
            ).wait()

    start_gather(0, 0)

    @pl.loop(0, NCH, step=2)
    def _chunks(g):
        for b in range(2):
            ch = g + b
            nb = (b + 1) % 2
            nxt = ch + 1

            @pl.when(nxt < NCH)
            def _():
                @pl.when(nxt >= 2)
                def _():
                    pltpu.make_async_copy(
                        rows[nb], out_hbm.at[pl.ds(0, CR)], ssem[nb]
                    ).wait()

                start_gather(nxt, nb)

            wait_gather(b)

            for j in range(CR):
                @pl.loop(0, COLS, unroll=4)
                def _scale(r):
                    for k in range(D // 16):
                        sl = pl.ds(k * 16, 16)
                        rows[b][j, r, sl] = rows[b][j, r, sl] * SCALE

            pltpu.async_copy(
                rows[b], out_hbm.at[pl.ds(base + ch * CR, CR)], ssem[b]
            )

    for b in range(2):
        pltpu.make_async_copy(rows[b], out_hbm.at[pl.ds(0, CR)], ssem[b]).wait()


_lookup = pl.kernel(
    _body,
    out_type=jax.ShapeDtypeStruct((ROWS, COLS, D), jnp.float32),
    mesh=_mesh,
    scratch_types=[
        pltpu.VMEM((CR, COLS), jnp.int32),
        pltpu.VMEM((CR, COLS), jnp.int32),
        pltpu.VMEM((CR, COLS, D), jnp.float32),
        pltpu.VMEM((CR, COLS, D), jnp.float32),
        pltpu.SemaphoreType.DMA,
        pltpu.SemaphoreType.DMA,
        pltpu.SemaphoreType.DMA,
        pltpu.SemaphoreType.DMA,
    ],
    compiler_params=pltpu.CompilerParams(use_tc_tiling_on_sc=False),
)


@jax.jit
def kernel(x, table):
    return _lookup(table, x)


# speedup vs baseline: 1.6198x; 1.6198x over previous
import math

import jax
import jax.numpy as jnp
from jax import lax
from jax.experimental import pallas as pl
from jax.experimental.pallas import tpu as pltpu
from jax.experimental.pallas import tpu_sc as plsc

VOCAB = 1000000
D = 64
ROWS = 16384
COLS = 200

NC = 2
NS = 16
NW = NC * NS
ROWS_W = ROWS // NW
CR = 2
NCH = ROWS_W // CR
CL = CR * COLS
SCALE = math.sqrt(D)

_mesh = plsc.VectorSubcoreMesh(
    core_axis_name="c", subcore_axis_name="s", num_cores=NC, num_subcores=NS
)


def _body(table_hbm, x_hbm, out_hbm, idx0, idx1, g0, g1,
          gsem0, gsem1, ssem0, ssem1):
    wid = lax.axis_index("s") * NC + lax.axis_index("c")
    base = wid * ROWS_W
    idx = [idx0, idx1]
    gbuf = [g0, g1]
    gsem = [gsem0, gsem1]
    ssem = [ssem0, ssem1]

    def start_gather(ch, b):
        r0 = base + ch * CR
        pltpu.sync_copy(x_hbm.at[pl.ds(r0, CR)], idx[b])
        for j in range(CR):
            pltpu.async_copy(
                table_hbm.at[idx[b].at[j]],
                gbuf[b].at[pl.ds(j * COLS, COLS)],
                gsem[b],
            )

    def wait_gather(b):
        for j in range(CR):
            pltpu.make_async_copy(
                table_hbm.at[idx[b].at[j]],
                gbuf[b].at[pl.ds(j * COLS, COLS)],
                gsem[b],
            ).wait()

    def start_scatter(ch, b):
        o0 = (base + ch * CR) * COLS
        pltpu.async_copy(
            gbuf[b], out_hbm.at[pl.ds(o0, CL), pl.ds(0, D)], ssem[b]
        )

    def wait_scatter(b):
        pltpu.make_async_copy(
            gbuf[b], out_hbm.at[pl.ds(0, CL), pl.ds(0, D)], ssem[b]
        ).wait()

    start_gather(0, 0)

    @pl.loop(0, NCH, step=2)
    def _chunks(g):
        for b in range(2):
            ch = g + b
            nb = (b + 1) % 2
            nxt = ch + 1

            @pl.when(nxt < NCH)
            def _():
                @pl.when(nxt >= 2)
                def _():
                    wait_scatter(nb)

                start_gather(nxt, nb)

            wait_gather(b)

            @pl.loop(0, CL, unroll=4)
            def _scale(r):
                for k in range(D // 16):
                    sl = pl.ds(k * 16, 16)
                    gbuf[b][r, sl] = gbuf[b][r, sl] * SCALE

            start_scatter(ch, b)

    for b in range(2):
        wait_scatter(b)


_lookup = pl.kernel(
    _body,
    out_type=jax.ShapeDtypeStruct((ROWS * COLS, 2 * D), jnp.float32),
    mesh=_mesh,
    scratch_types=[
        pltpu.VMEM((CR, COLS), jnp.int32),
        pltpu.VMEM((CR, COLS), jnp.int32),
        pltpu.VMEM((CL, D), jnp.float32),
        pltpu.VMEM((CL, D), jnp.float32),
        pltpu.SemaphoreType.DMA,
        pltpu.SemaphoreType.DMA,
        pltpu.SemaphoreType.DMA,
        pltpu.SemaphoreType.DMA,
    ],
    compiler_params=pltpu.CompilerParams(use_tc_tiling_on_sc=False),
)


@jax.jit
def kernel(x, table):
    out = _lookup(table, x)
    return out[:, :D].reshape(ROWS, COLS, D)


# speedup vs baseline: 1.7028x; 1.0512x over previous
import math

import jax
import jax.numpy as jnp
from jax import lax
from jax.experimental import pallas as pl
from jax.experimental.pallas import tpu as pltpu
from jax.experimental.pallas import tpu_sc as plsc

VOCAB = 1000000
D = 64
ROWS = 16384
COLS = 200

NC = 2
NS = 16
NW = NC * NS
ROWS_W = ROWS // NW
CR = 4
NCH = ROWS_W // CR
CL = CR * COLS
SCALE = math.sqrt(D)

_mesh = plsc.VectorSubcoreMesh(
    core_axis_name="c", subcore_axis_name="s", num_cores=NC, num_subcores=NS
)


def _body(table_hbm, x_hbm, out_hbm, idx0, idx1, g0, g1,
          isem0, isem1, gsem0, gsem1, ssem0, ssem1):
    wid = lax.axis_index("s") * NC + lax.axis_index("c")
    base = wid * ROWS_W
    idx = [idx0, idx1]
    gbuf = [g0, g1]
    isem = [isem0, isem1]
    gsem = [gsem0, gsem1]
    ssem = [ssem0, ssem1]

    def start_idx(ch, b):
        pltpu.async_copy(
            x_hbm.at[pl.ds(base + ch * CR, CR)], idx[b], isem[b]
        )

    def wait_idx(b):
        pltpu.make_async_copy(
            x_hbm.at[pl.ds(0, CR)], idx[b], isem[b]
        ).wait()

    def start_gather(b):
        for j in range(CR):
            pltpu.async_copy(
                table_hbm.at[idx[b].at[j]],
                gbuf[b].at[pl.ds(j * COLS, COLS)],
                gsem[b],
            )

    def wait_gather(b):
        for j in range(CR):
            pltpu.make_async_copy(
                table_hbm.at[idx[b].at[j]],
                gbuf[b].at[pl.ds(j * COLS, COLS)],
                gsem[b],
            ).wait()

    def start_scatter(ch, b):
        o0 = (base + ch * CR) * COLS
        pltpu.async_copy(
            gbuf[b], out_hbm.at[pl.ds(o0, CL), pl.ds(0, D)], ssem[b]
        )

    def wait_scatter(b):
        pltpu.make_async_copy(
            gbuf[b], out_hbm.at[pl.ds(0, CL), pl.ds(0, D)], ssem[b]
        ).wait()

    start_idx(0, 0)
    start_idx(1, 1)
    wait_idx(0)
    start_gather(0)

    @pl.loop(0, NCH, step=2)
    def _chunks(g):
        for b in range(2):
            ch = g + b
            nb = (b + 1) % 2
            nxt = ch + 1

            @pl.when(nxt < NCH)
            def _():
                wait_idx(nb)

                @pl.when(nxt >= 2)
                def _():
                    wait_scatter(nb)

                start_gather(nb)

            wait_gather(b)

            @pl.when(ch + 2 < NCH)
            def _():
                start_idx(ch + 2, b)

            @pl.loop(0, CL, unroll=8)
            def _scale(r):
                for k in range(D // 16):
                    sl = pl.ds(k * 16, 16)
                    gbuf[b][r, sl] = gbuf[b][r, sl] * SCALE

            start_scatter(ch, b)

    for b in range(2):
        wait_scatter(b)


_lookup = pl.kernel(
    _body,
    out_type=jax.ShapeDtypeStruct((ROWS * COLS, 2 * D), jnp.float32),
    mesh=_mesh,
    scratch_types=[
        pltpu.VMEM((CR, COLS), jnp.int32),
        pltpu.VMEM((CR, COLS), jnp.int32),
        pltpu.VMEM((CL, D), jnp.float32),
        pltpu.VMEM((CL, D), jnp.float32),
        pltpu.SemaphoreType.DMA,
        pltpu.SemaphoreType.DMA,
        pltpu.SemaphoreType.DMA,
        pltpu.SemaphoreType.DMA,
        pltpu.SemaphoreType.DMA,
        pltpu.SemaphoreType.DMA,
    ],
    compiler_params=pltpu.CompilerParams(use_tc_tiling_on_sc=False),
)


@jax.jit
def kernel(x, table):
    out = _lookup(table, x)
    return out[:, :D].reshape(ROWS, COLS, D)
